# baseline (speedup 1.0000x reference)
import functools

import jax
import jax.numpy as jnp
from jax import lax
from jax.experimental import pallas as pl
from jax.experimental.pallas import tpu as pltpu
from jax.experimental.pallas import tpu_sc as plsc

B = 16384
F = 26
K = 16
L = 16
NC = 2
NS = 16
NW = NC * NS
BPW = B // NW
C = 64
NCHUNK = BPW // C
G = C * F
PLANE = 2600000


def _fm_body(x_hbm, emb_hbm, fc_hbm, lw_hbm, lb_hbm, out_hbm,
             idx_v, pidx_v, vals_v, fc_v, out_v, lw_s, lb_s, sem_e, sem_f):
    wid = lax.axis_index("s") * NC + lax.axis_index("c")
    ibase = wid * (BPW * F)

    pltpu.sync_copy(x_hbm.at[pl.ds(ibase, BPW * F)], idx_v)
    pltpu.sync_copy(lw_hbm, lw_s)
    pltpu.sync_copy(lb_hbm, lb_s)

    lw = lw_s[...]
    lb = lb_s[...]
    iota = lax.iota(jnp.int32, L)

    def chunk_body(c, carry):
        base = c * G
        fc_dma = pltpu.async_copy(fc_hbm.at[idx_v.at[pl.ds(base, G)]],
                                  fc_v, sem_f)

        def pidx_body(a, carry2):
            vi = idx_v[pl.ds(base + a * L, L)]
            for k in range(K):
                pidx_v[pl.ds(k * G + a * L, L)] = vi + k * PLANE
            return carry2

        lax.fori_loop(0, G // L, pidx_body, 0)

        dmas = []
        for k in range(K):
            dmas.append(pltpu.async_copy(
                emb_hbm.at[pidx_v.at[pl.ds(k * G, G)]],
                vals_v.at[pl.ds(k * G, G)], sem_e))
        for d in dmas:
            d.wait()
        fc_dma.wait()

        for g in range(C // L):
            rowbase = (g * L + iota) * F
            tot = None
            for k in range(K):
                s = plsc.load_gather(vals_v, [rowbase + k * G])
                q = s * s
                for f in range(1, F):
                    v = plsc.load_gather(vals_v, [rowbase + (k * G + f)])
                    s = s + v
                    q = q + v * v
                contrib = s * s - q
                tot = contrib if tot is None else tot + contrib
            fs = plsc.load_gather(fc_v, [rowbase])
            for f in range(1, F):
                fs = fs + plsc.load_gather(fc_v, [rowbase + f])
            z = lw * fs + lb + 0.5 * tot
            out_v[pl.ds(c * C + g * L, L)] = 1.0 / (1.0 + jnp.exp(-z))
        return carry

    lax.fori_loop(0, NCHUNK, chunk_body, 0)
    pltpu.sync_copy(out_v, out_hbm.at[pl.ds(wid * BPW, BPW)])


_fm_kernel = functools.partial(
    pl.kernel,
    out_type=jax.ShapeDtypeStruct((B,), jnp.float32),
    mesh=plsc.VectorSubcoreMesh(core_axis_name="c", subcore_axis_name="s"),
    compiler_params=pltpu.CompilerParams(
        needs_layout_passes=False, use_tc_tiling_on_sc=False),
    scratch_types=[
        pltpu.VMEM((BPW * F,), jnp.int32),
        pltpu.VMEM((K * G,), jnp.int32),
        pltpu.VMEM((K * G,), jnp.float32),
        pltpu.VMEM((G,), jnp.float32),
        pltpu.VMEM((BPW,), jnp.float32),
        pltpu.VMEM((L,), jnp.float32),
        pltpu.VMEM((L,), jnp.float32),
        pltpu.SemaphoreType.DMA,
        pltpu.SemaphoreType.DMA,
    ],
)(_fm_body)


def kernel(x, emb_table, fc_table, lin_w, lin_b, W0, b0, W1, b1, W2, b2):
    xf = x.reshape(B * F)
    emb_flat = emb_table.T.reshape(K * PLANE)
    lw16 = jnp.broadcast_to(lin_w.reshape(1), (L,))
    lb16 = jnp.broadcast_to(lin_b.reshape(1), (L,))
    return _fm_kernel(xf, emb_flat, fc_table[:, 0], lw16, lb16)


# speedup vs baseline: 3.8467x; 3.8467x over previous
import functools

import jax
import jax.numpy as jnp
from jax import lax
from jax.experimental import pallas as pl
from jax.experimental.pallas import tpu as pltpu
from jax.experimental.pallas import tpu_sc as plsc

B = 16384
F = 26
K = 16
L = 16
NC = 2
NS = 16
NW = NC * NS
BPW = B // NW
C = 128
NCHUNK = BPW // C
G = C * F
PLANE = 2600000
TI = 8192
NTI = -(-PLANE // TI)


def _tr_body(in_ref, out_ref, t_ref):
    t_ref[...] = in_ref[...].T
    for j in range(8):
        out_ref[:, j * K:(j + 1) * K] = t_ref[pl.Slice(j, TI // 8, 8), :]


_tr_kernel = pl.pallas_call(
    _tr_body,
    grid=(NTI,),
    in_specs=[pl.BlockSpec((K, TI), lambda j: (0, j))],
    out_specs=pl.BlockSpec((TI // 8, 128), lambda j: (j, 0)),
    out_shape=jax.ShapeDtypeStruct((PLANE // 8, 128), jnp.float32),
    scratch_shapes=[pltpu.VMEM((TI, K), jnp.float32)],
)


def _fm_body(x_hbm, emb_hbm, fc_hbm, lw_hbm, lb_hbm, out_hbm,
             idx_v, fc_v, rows_v, u_v, out_v, lw_s, lb_s, sem_e, sem_f):
    wid = lax.axis_index("s") * NC + lax.axis_index("c")
    ibase = wid * (BPW * F)

    pltpu.sync_copy(x_hbm.at[pl.ds(ibase, BPW * F)], idx_v)
    pltpu.sync_copy(lw_hbm, lw_s)
    pltpu.sync_copy(lb_hbm, lb_s)

    lw = lw_s[...]
    lb = lb_s[...]
    iota = lax.iota(jnp.int32, L)

    def chunk_body(c, carry):
        idx_slice = idx_v.at[pl.ds(c * G, G)]
        fc_dma = pltpu.async_copy(fc_hbm.at[idx_slice], fc_v, sem_f)
        pltpu.async_copy(emb_hbm.at[idx_slice], rows_v, sem_e).wait()

        def batch_body(b, carry2):
            off = b * F
            acc = rows_v[off]
            acc2 = acc * acc
            for f in range(1, F):
                v = rows_v[off + f]
                acc = acc + v
                acc2 = acc2 + v * v
            u_v[pl.ds(b * K, K)] = acc * acc - acc2
            return carry2

        lax.fori_loop(0, C, batch_body, 0)
        fc_dma.wait()

        def group_body(g, carry2):
            rowids = g * L + iota
            t = plsc.load_gather(u_v, [rowids * K])
            for k in range(1, K):
                t = t + plsc.load_gather(u_v, [rowids * K + k])
            fs = plsc.load_gather(fc_v, [rowids * F])
            for f in range(1, F):
                fs = fs + plsc.load_gather(fc_v, [rowids * F + f])
            z = lw * fs + lb + 0.5 * t
            out_v[pl.ds(c * C + g * L, L)] = 1.0 / (1.0 + jnp.exp(-z))
            return carry2

        lax.fori_loop(0, C // L, group_body, 0)
        return carry

    lax.fori_loop(0, NCHUNK, chunk_body, 0)
    pltpu.sync_copy(out_v, out_hbm.at[pl.ds(wid * BPW, BPW)])


_fm_kernel = functools.partial(
    pl.kernel,
    out_type=jax.ShapeDtypeStruct((B,), jnp.float32),
    mesh=plsc.VectorSubcoreMesh(core_axis_name="c", subcore_axis_name="s"),
    compiler_params=pltpu.CompilerParams(
        needs_layout_passes=False, use_tc_tiling_on_sc=False),
    scratch_types=[
        pltpu.VMEM((BPW * F,), jnp.int32),
        pltpu.VMEM((G,), jnp.float32),
        pltpu.VMEM((G, K), jnp.float32),
        pltpu.VMEM((C * K,), jnp.float32),
        pltpu.VMEM((BPW,), jnp.float32),
        pltpu.VMEM((L,), jnp.float32),
        pltpu.VMEM((L,), jnp.float32),
        pltpu.SemaphoreType.DMA,
        pltpu.SemaphoreType.DMA,
    ],
)(_fm_body)


def kernel(x, emb_table, fc_table, lin_w, lin_b, W0, b0, W1, b1, W2, b2):
    xf = x.reshape(B * F)
    emb_rm = _tr_kernel(emb_table.T).reshape(PLANE, K)
    lw16 = jnp.broadcast_to(lin_w.reshape(1), (L,))
    lb16 = jnp.broadcast_to(lin_b.reshape(1), (L,))
    return _fm_kernel(xf, emb_rm, fc_table[:, 0], lw16, lb16)


# speedup vs baseline: 8.1788x; 2.1262x over previous
import functools

import jax
import jax.numpy as jnp
from jax import lax
from jax.experimental import pallas as pl
from jax.experimental.pallas import tpu as pltpu
from jax.experimental.pallas import tpu_sc as plsc

B = 16384
F = 26
K = 16
L = 16
NC = 2
NS = 16
NW = NC * NS
BPW = B // NW
C = 128
NCHUNK = BPW // C
G = C * F
PLANE = 2600000
TI = 8192
NTI = -(-PLANE // TI)
OUTR = NTI * (TI // 8)


def _tr_body(in_ref, out_ref):
    for m in range(TI // 1024):
        base = m * 1024
        x = jnp.concatenate(
            [in_ref[:, base + u * 128: base + (u + 1) * 128]
             for u in range(8)], axis=0)
        out_ref[m * 128:(m + 1) * 128, :] = x.T


_tr_kernel = pl.pallas_call(
    _tr_body,
    grid=(NTI,),
    in_specs=[pl.BlockSpec((K, TI), lambda j: (0, j))],
    out_specs=pl.BlockSpec((TI // 8, 128), lambda j: (j, 0)),
    out_shape=jax.ShapeDtypeStruct((OUTR, 128), jnp.float32),
)


def _fm_body(x_hbm, emb_hbm, fc_hbm, lw_hbm, lb_hbm, out_hbm,
             idx_v, tix_v, fc_v, rows_v, u_v, out_v, lw_s, lb_s,
             sem_e, sem_f):
    wid = lax.axis_index("s") * NC + lax.axis_index("c")
    ibase = wid * (BPW * F)

    pltpu.sync_copy(x_hbm.at[pl.ds(ibase, BPW * F)], idx_v)
    pltpu.sync_copy(lw_hbm, lw_s)
    pltpu.sync_copy(lb_hbm, lb_s)

    lw = lw_s[...]
    lb = lb_s[...]
    iota = lax.iota(jnp.int32, L)

    def chunk_body(c, carry):
        idx_slice = idx_v.at[pl.ds(c * G, G)]
        fc_dma = pltpu.async_copy(fc_hbm.at[idx_slice], fc_v, sem_f)

        def tix_body(i, carry3):
            p = idx_v[pl.ds(c * G + i * L, L)]
            low = p & 1023
            tix_v[pl.ds(i * L, L)] = p - low + ((p & 127) << 3) + (low >> 7)
            return carry3

        lax.fori_loop(0, G // L, tix_body, 0)
        pltpu.async_copy(emb_hbm.at[tix_v], rows_v, sem_e).wait()

        def batch_body(b, carry2):
            off = b * F
            acc = rows_v[off]
            acc2 = acc * acc
            for f in range(1, F):
                v = rows_v[off + f]
                acc = acc + v
                acc2 = acc2 + v * v
            u_v[pl.ds(b * K, K)] = acc * acc - acc2
            return carry2

        lax.fori_loop(0, C, batch_body, 0)
        fc_dma.wait()

        def group_body(g, carry2):
            rowids = g * L + iota
            t = plsc.load_gather(u_v, [rowids * K])
            for k in range(1, K):
                t = t + plsc.load_gather(u_v, [rowids * K + k])
            fs = plsc.load_gather(fc_v, [rowids * F])
            for f in range(1, F):
                fs = fs + plsc.load_gather(fc_v, [rowids * F + f])
            z = lw * fs + lb + 0.5 * t
            out_v[pl.ds(c * C + g * L, L)] = 1.0 / (1.0 + jnp.exp(-z))
            return carry2

        lax.fori_loop(0, C // L, group_body, 0)
        return carry

    lax.fori_loop(0, NCHUNK, chunk_body, 0)
    pltpu.sync_copy(out_v, out_hbm.at[pl.ds(wid * BPW, BPW)])


_fm_kernel = functools.partial(
    pl.kernel,
    out_type=jax.ShapeDtypeStruct((B,), jnp.float32),
    mesh=plsc.VectorSubcoreMesh(core_axis_name="c", subcore_axis_name="s"),
    compiler_params=pltpu.CompilerParams(
        needs_layout_passes=False, use_tc_tiling_on_sc=False),
    scratch_types=[
        pltpu.VMEM((BPW * F,), jnp.int32),
        pltpu.VMEM((G,), jnp.int32),
        pltpu.VMEM((G,), jnp.float32),
        pltpu.VMEM((G, K), jnp.float32),
        pltpu.VMEM((C * K,), jnp.float32),
        pltpu.VMEM((BPW,), jnp.float32),
        pltpu.VMEM((L,), jnp.float32),
        pltpu.VMEM((L,), jnp.float32),
        pltpu.SemaphoreType.DMA,
        pltpu.SemaphoreType.DMA,
    ],
)(_fm_body)


def kernel(x, emb_table, fc_table, lin_w, lin_b, W0, b0, W1, b1, W2, b2):
    xf = x.reshape(B * F)
    emb_rm = _tr_kernel(emb_table.T).reshape(OUTR * 8, K)
    lw16 = jnp.broadcast_to(lin_w.reshape(1), (L,))
    lb16 = jnp.broadcast_to(lin_b.reshape(1), (L,))
    return _fm_kernel(xf, emb_rm, fc_table[:, 0], lw16, lb16)


# speedup vs baseline: 11.2377x; 1.3740x over previous
import functools

import jax
import jax.numpy as jnp
from jax import lax
from jax.experimental import pallas as pl
from jax.experimental.pallas import tpu as pltpu
from jax.experimental.pallas import tpu_sc as plsc

B = 16384
F = 26
K = 16
L = 16
NC = 2
NS = 16
NW = NC * NS
BPW = B // NW
C = 128
NCHUNK = BPW // C
G = C * F
PLANE = 2600000
TI = 32768
NTI = -(-PLANE // TI)
OUTR = NTI * (TI // 8)


def _tr_body(in_ref, out_ref):
    for m in range(TI // 1024):
        base = m * 1024
        x = jnp.concatenate(
            [in_ref[:, base + u * 128: base + (u + 1) * 128]
             for u in range(8)], axis=0)
        out_ref[m * 128:(m + 1) * 128, :] = x.T


_tr_kernel = pl.pallas_call(
    _tr_body,
    grid=(NTI,),
    in_specs=[pl.BlockSpec((K, TI), lambda j: (0, j))],
    out_specs=pl.BlockSpec((TI // 8, 128), lambda j: (j, 0)),
    out_shape=jax.ShapeDtypeStruct((OUTR, 128), jnp.float32),
)


def _fm_body(x_hbm, emb_hbm, fc_hbm, lw_hbm, lb_hbm, out_hbm,
             idx_v, tix_v, fc_v, rows_v, u_v, out_v, lw_s, lb_s,
             sem_e, sem_f):
    wid = lax.axis_index("s") * NC + lax.axis_index("c")
    ibase = wid * (BPW * F)

    pltpu.sync_copy(x_hbm.at[pl.ds(ibase, BPW * F)], idx_v)
    pltpu.sync_copy(lw_hbm, lw_s)
    pltpu.sync_copy(lb_hbm, lb_s)

    lw = lw_s[...]
    lb = lb_s[...]
    iota = lax.iota(jnp.int32, L)

    def chunk_body(c, carry):
        idx_slice = idx_v.at[pl.ds(c * G, G)]
        fc_dma = pltpu.async_copy(fc_hbm.at[idx_slice], fc_v, sem_f)

        def tix_body(i, carry3):
            p = idx_v[pl.ds(c * G + i * L, L)]
            low = p & 1023
            tix_v[pl.ds(i * L, L)] = p - low + ((p & 127) << 3) + (low >> 7)
            return carry3

        lax.fori_loop(0, G // L, tix_body, 0)
        pltpu.async_copy(emb_hbm.at[tix_v], rows_v, sem_e).wait()

        def batch_body(b, carry2):
            off = b * F
            acc = rows_v[off]
            acc2 = acc * acc
            for f in range(1, F):
                v = rows_v[off + f]
                acc = acc + v
                acc2 = acc2 + v * v
            u_v[pl.ds(b * K, K)] = acc * acc - acc2
            return carry2

        lax.fori_loop(0, C, batch_body, 0)
        fc_dma.wait()

        def group_body(g, carry2):
            rowids = g * L + iota
            t = plsc.load_gather(u_v, [rowids * K])
            for k in range(1, K):
                t = t + plsc.load_gather(u_v, [rowids * K + k])
            fs = plsc.load_gather(fc_v, [rowids * F])
            for f in range(1, F):
                fs = fs + plsc.load_gather(fc_v, [rowids * F + f])
            z = lw * fs + lb + 0.5 * t
            out_v[pl.ds(c * C + g * L, L)] = 1.0 / (1.0 + jnp.exp(-z))
            return carry2

        lax.fori_loop(0, C // L, group_body, 0)
        return carry

    lax.fori_loop(0, NCHUNK, chunk_body, 0)
    pltpu.sync_copy(out_v, out_hbm.at[pl.ds(wid * BPW, BPW)])


_fm_kernel = functools.partial(
    pl.kernel,
    out_type=jax.ShapeDtypeStruct((B,), jnp.float32),
    mesh=plsc.VectorSubcoreMesh(core_axis_name="c", subcore_axis_name="s"),
    compiler_params=pltpu.CompilerParams(
        needs_layout_passes=False, use_tc_tiling_on_sc=False),
    scratch_types=[
        pltpu.VMEM((BPW * F,), jnp.int32),
        pltpu.VMEM((G,), jnp.int32),
        pltpu.VMEM((G,), jnp.float32),
        pltpu.VMEM((G, K), jnp.float32),
        pltpu.VMEM((C * K,), jnp.float32),
        pltpu.VMEM((BPW,), jnp.float32),
        pltpu.VMEM((L,), jnp.float32),
        pltpu.VMEM((L,), jnp.float32),
        pltpu.SemaphoreType.DMA,
        pltpu.SemaphoreType.DMA,
    ],
)(_fm_body)


def kernel(x, emb_table, fc_table, lin_w, lin_b, W0, b0, W1, b1, W2, b2):
    xf = x.reshape(B * F)
    emb_rm = _tr_kernel(emb_table.T).reshape(OUTR * 8, K)
    lw16 = jnp.broadcast_to(lin_w.reshape(1), (L,))
    lb16 = jnp.broadcast_to(lin_b.reshape(1), (L,))
    return _fm_kernel(xf, emb_rm, fc_table[:, 0], lw16, lb16)


# speedup vs baseline: 11.8834x; 1.0575x over previous
import functools

import jax
import jax.numpy as jnp
from jax import lax
from jax.experimental import pallas as pl
from jax.experimental.pallas import tpu as pltpu
from jax.experimental.pallas import tpu_sc as plsc

B = 16384
F = 26
K = 16
L = 16
NC = 2
NS = 16
NW = NC * NS
BPW = B // NW
C = 128
NCHUNK = BPW // C
G = C * F
PLANE = 2600000
TI = 65536
NTI = -(-PLANE // TI)
OUTR = NTI * (TI // 8)


def _tr_body(in_ref, out_ref):
    for m in range(TI // 1024):
        base = m * 1024
        x = jnp.concatenate(
            [in_ref[:, base + u * 128: base + (u + 1) * 128]
             for u in range(8)], axis=0)
        out_ref[m * 128:(m + 1) * 128, :] = x.T


_tr_kernel = pl.pallas_call(
    _tr_body,
    grid=(NTI,),
    in_specs=[pl.BlockSpec((K, TI), lambda j: (0, j))],
    out_specs=pl.BlockSpec((TI // 8, 128), lambda j: (j, 0)),
    out_shape=jax.ShapeDtypeStruct((OUTR, 128), jnp.float32),
)


def _fm_body(x_hbm, emb_hbm, fc_hbm, lw_hbm, lb_hbm, out_hbm,
             idx_v, tix_v, fc_v, rows_v, u_v, out_v, lw_s, lb_s,
             sem_e, sem_f):
    wid = lax.axis_index("s") * NC + lax.axis_index("c")
    ibase = wid * (BPW * F)

    pltpu.sync_copy(x_hbm.at[pl.ds(ibase, BPW * F)], idx_v)
    pltpu.sync_copy(lw_hbm, lw_s)
    pltpu.sync_copy(lb_hbm, lb_s)

    lw = lw_s[...]
    lb = lb_s[...]
    iota = lax.iota(jnp.int32, L)

    def chunk_body(c, carry):
        idx_slice = idx_v.at[pl.ds(c * G, G)]
        fc_dma = pltpu.async_copy(fc_hbm.at[idx_slice], fc_v, sem_f)

        def tix_body(i, carry3):
            p = idx_v[pl.ds(c * G + i * L, L)]
            low = p & 1023
            tix_v[pl.ds(i * L, L)] = p - low + ((p & 127) << 3) + (low >> 7)
            return carry3

        lax.fori_loop(0, G // L, tix_body, 0)
        pltpu.async_copy(emb_hbm.at[tix_v], rows_v, sem_e).wait()

        def batch_body(b, carry2):
            off = b * F
            acc = rows_v[off]
            acc2 = acc * acc
            for f in range(1, F):
                v = rows_v[off + f]
                acc = acc + v
                acc2 = acc2 + v * v
            u_v[pl.ds(b * K, K)] = acc * acc - acc2
            return carry2

        lax.fori_loop(0, C, batch_body, 0)
        fc_dma.wait()

        def group_body(g, carry2):
            rowids = g * L + iota
            t = plsc.load_gather(u_v, [rowids * K])
            for k in range(1, K):
                t = t + plsc.load_gather(u_v, [rowids * K + k])
            fs = plsc.load_gather(fc_v, [rowids * F])
            for f in range(1, F):
                fs = fs + plsc.load_gather(fc_v, [rowids * F + f])
            z = lw * fs + lb + 0.5 * t
            out_v[pl.ds(c * C + g * L, L)] = 1.0 / (1.0 + jnp.exp(-z))
            return carry2

        lax.fori_loop(0, C // L, group_body, 0)
        return carry

    lax.fori_loop(0, NCHUNK, chunk_body, 0)
    pltpu.sync_copy(out_v, out_hbm.at[pl.ds(wid * BPW, BPW)])


_fm_kernel = functools.partial(
    pl.kernel,
    out_type=jax.ShapeDtypeStruct((B,), jnp.float32),
    mesh=plsc.VectorSubcoreMesh(core_axis_name="c", subcore_axis_name="s"),
    compiler_params=pltpu.CompilerParams(
        needs_layout_passes=False, use_tc_tiling_on_sc=False),
    scratch_types=[
        pltpu.VMEM((BPW * F,), jnp.int32),
        pltpu.VMEM((G,), jnp.int32),
        pltpu.VMEM((G,), jnp.float32),
        pltpu.VMEM((G, K), jnp.float32),
        pltpu.VMEM((C * K,), jnp.float32),
        pltpu.VMEM((BPW,), jnp.float32),
        pltpu.VMEM((L,), jnp.float32),
        pltpu.VMEM((L,), jnp.float32),
        pltpu.SemaphoreType.DMA,
        pltpu.SemaphoreType.DMA,
    ],
)(_fm_body)


def kernel(x, emb_table, fc_table, lin_w, lin_b, W0, b0, W1, b1, W2, b2):
    xf = x.reshape(B * F)
    emb_rm = _tr_kernel(emb_table.T).reshape(OUTR * 8, K)
    lw16 = jnp.broadcast_to(lin_w.reshape(1), (L,))
    lb16 = jnp.broadcast_to(lin_b.reshape(1), (L,))
    return _fm_kernel(xf, emb_rm, fc_table[:, 0], lw16, lb16)


# speedup vs baseline: 11.9813x; 1.0082x over previous
import functools

import jax
import jax.numpy as jnp
from jax import lax
from jax.experimental import pallas as pl
from jax.experimental.pallas import tpu as pltpu
from jax.experimental.pallas import tpu_sc as plsc

B = 16384
F = 26
K = 16
L = 16
NC = 2
NS = 16
NW = NC * NS
BPW = B // NW
C = 128
NCHUNK = BPW // C
G = C * F
PLANE = 2600000
TI = 131072
NTI = -(-PLANE // TI)
OUTR = NTI * (TI // 8)


def _tr_body(in_ref, out_ref):
    for m in range(TI // 1024):
        base = m * 1024
        x = jnp.concatenate(
            [in_ref[:, base + u * 128: base + (u + 1) * 128]
             for u in range(8)], axis=0)
        out_ref[m * 128:(m + 1) * 128, :] = x.T


_tr_kernel = pl.pallas_call(
    _tr_body,
    grid=(NTI,),
    in_specs=[pl.BlockSpec((K, TI), lambda j: (0, j))],
    out_specs=pl.BlockSpec((TI // 8, 128), lambda j: (j, 0)),
    out_shape=jax.ShapeDtypeStruct((OUTR, 128), jnp.float32),
)


def _fm_body(x_hbm, emb_hbm, fc_hbm, lw_hbm, lb_hbm, out_hbm,
             idx_v, tix_v, fc_v, rows_v, u_v, out_v, lw_s, lb_s,
             sem_e, sem_f):
    wid = lax.axis_index("s") * NC + lax.axis_index("c")
    ibase = wid * (BPW * F)

    pltpu.sync_copy(x_hbm.at[pl.ds(ibase, BPW * F)], idx_v)
    pltpu.sync_copy(lw_hbm, lw_s)
    pltpu.sync_copy(lb_hbm, lb_s)

    lw = lw_s[...]
    lb = lb_s[...]
    iota = lax.iota(jnp.int32, L)

    def chunk_body(c, carry):
        idx_slice = idx_v.at[pl.ds(c * G, G)]
        fc_dma = pltpu.async_copy(fc_hbm.at[idx_slice], fc_v, sem_f)

        def tix_body(i, carry3):
            p = idx_v[pl.ds(c * G + i * L, L)]
            low = p & 1023
            tix_v[pl.ds(i * L, L)] = p - low + ((p & 127) << 3) + (low >> 7)
            return carry3

        lax.fori_loop(0, G // L, tix_body, 0)
        pltpu.async_copy(emb_hbm.at[tix_v], rows_v, sem_e).wait()

        def batch_body(b, carry2):
            off = b * F
            acc = rows_v[off]
            acc2 = acc * acc
            for f in range(1, F):
                v = rows_v[off + f]
                acc = acc + v
                acc2 = acc2 + v * v
            u_v[pl.ds(b * K, K)] = acc * acc - acc2
            return carry2

        lax.fori_loop(0, C, batch_body, 0)
        fc_dma.wait()

        def group_body(g, carry2):
            rowids = g * L + iota
            t = plsc.load_gather(u_v, [rowids * K])
            for k in range(1, K):
                t = t + plsc.load_gather(u_v, [rowids * K + k])
            fs = plsc.load_gather(fc_v, [rowids * F])
            for f in range(1, F):
                fs = fs + plsc.load_gather(fc_v, [rowids * F + f])
            z = lw * fs + lb + 0.5 * t
            out_v[pl.ds(c * C + g * L, L)] = 1.0 / (1.0 + jnp.exp(-z))
            return carry2

        lax.fori_loop(0, C // L, group_body, 0)
        return carry

    lax.fori_loop(0, NCHUNK, chunk_body, 0)
    pltpu.sync_copy(out_v, out_hbm.at[pl.ds(wid * BPW, BPW)])


_fm_kernel = functools.partial(
    pl.kernel,
    out_type=jax.ShapeDtypeStruct((B,), jnp.float32),
    mesh=plsc.VectorSubcoreMesh(core_axis_name="c", subcore_axis_name="s"),
    compiler_params=pltpu.CompilerParams(
        needs_layout_passes=False, use_tc_tiling_on_sc=False),
    scratch_types=[
        pltpu.VMEM((BPW * F,), jnp.int32),
        pltpu.VMEM((G,), jnp.int32),
        pltpu.VMEM((G,), jnp.float32),
        pltpu.VMEM((G, K), jnp.float32),
        pltpu.VMEM((C * K,), jnp.float32),
        pltpu.VMEM((BPW,), jnp.float32),
        pltpu.VMEM((L,), jnp.float32),
        pltpu.VMEM((L,), jnp.float32),
        pltpu.SemaphoreType.DMA,
        pltpu.SemaphoreType.DMA,
    ],
)(_fm_body)


def kernel(x, emb_table, fc_table, lin_w, lin_b, W0, b0, W1, b1, W2, b2):
    xf = x.reshape(B * F)
    emb_rm = _tr_kernel(emb_table.T).reshape(OUTR * 8, K)
    lw16 = jnp.broadcast_to(lin_w.reshape(1), (L,))
    lb16 = jnp.broadcast_to(lin_b.reshape(1), (L,))
    return _fm_kernel(xf, emb_rm, fc_table[:, 0], lw16, lb16)


# speedup vs baseline: 12.5605x; 1.0483x over previous
import functools

import jax
import jax.numpy as jnp
from jax import lax
from jax.experimental import pallas as pl
from jax.experimental.pallas import tpu as pltpu
from jax.experimental.pallas import tpu_sc as plsc

B = 16384
F = 26
K = 16
L = 16
NC = 2
NS = 16
NW = NC * NS
BPW = B // NW
C = 64
NCHUNK = BPW // C
G = C * F
PLANE = 2600000
TI = 131072
NTI = -(-PLANE // TI)
OUTR = NTI * (TI // 8)


def _tr_body(in_ref, out_ref):
    for m in range(TI // 1024):
        base = m * 1024
        x = jnp.concatenate(
            [in_ref[:, base + u * 128: base + (u + 1) * 128]
             for u in range(8)], axis=0)
        out_ref[m * 128:(m + 1) * 128, :] = x.T


_tr_kernel = pl.pallas_call(
    _tr_body,
    grid=(NTI,),
    in_specs=[pl.BlockSpec((K, TI), lambda j: (0, j))],
    out_specs=pl.BlockSpec((TI // 8, 128), lambda j: (j, 0)),
    out_shape=jax.ShapeDtypeStruct((OUTR, 128), jnp.float32),
)


def _fm_body(x_hbm, emb_hbm, fc_hbm, lw_hbm, lb_hbm, out_hbm,
             idx_v, tix_a, tix_b, fc_a, fc_b, rows_a, rows_b,
             u_v, out_v, lw_s, lb_s, se_a, se_b, sf_a, sf_b):
    wid = lax.axis_index("s") * NC + lax.axis_index("c")
    ibase = wid * (BPW * F)

    pltpu.sync_copy(x_hbm.at[pl.ds(ibase, BPW * F)], idx_v)
    pltpu.sync_copy(lw_hbm, lw_s)
    pltpu.sync_copy(lb_hbm, lb_s)

    lw = lw_s[...]
    lb = lb_s[...]
    iota = lax.iota(jnp.int32, L)
    bufs = [(tix_a, fc_a, rows_a, se_a, sf_a),
            (tix_b, fc_b, rows_b, se_b, sf_b)]

    def start_gathers(c, tix_v, fc_v, rows_v, sem_e, sem_f):
        fc_dma = pltpu.async_copy(fc_hbm.at[idx_v.at[pl.ds(c * G, G)]],
                                  fc_v, sem_f)

        def tix_body(i, carry3):
            p = idx_v[pl.ds(c * G + i * L, L)]
            low = p & 1023
            tix_v[pl.ds(i * L, L)] = p - low + ((p & 127) << 3) + (low >> 7)
            return carry3

        lax.fori_loop(0, G // L, tix_body, 0)
        e_dma = pltpu.async_copy(emb_hbm.at[tix_v], rows_v, sem_e)
        return e_dma, fc_dma

    def compute(c, fc_v, rows_v):
        def batch_body(b, carry2):
            off = b * F
            acc = rows_v[off]
            acc2 = acc * acc
            for f in range(1, F):
                v = rows_v[off + f]
                acc = acc + v
                acc2 = acc2 + v * v
            u_v[pl.ds(b * K, K)] = acc * acc - acc2
            return carry2

        lax.fori_loop(0, C, batch_body, 0)

        def group_body(g, carry2):
            rowids = g * L + iota
            t = plsc.load_gather(u_v, [rowids * K])
            for k in range(1, K):
                t = t + plsc.load_gather(u_v, [rowids * K + k])
            fs = plsc.load_gather(fc_v, [rowids * F])
            for f in range(1, F):
                fs = fs + plsc.load_gather(fc_v, [rowids * F + f])
            z = lw * fs + lb + 0.5 * t
            out_v[pl.ds(c * C + g * L, L)] = 1.0 / (1.0 + jnp.exp(-z))
            return carry2

        lax.fori_loop(0, C // L, group_body, 0)

    dmas = start_gathers(0, *bufs[0])
    for c in range(NCHUNK):
        tix_c, fc_c, rows_c, _, _ = bufs[c % 2]
        e_dma, fc_dma = dmas
        if c + 1 < NCHUNK:
            dmas = start_gathers(c + 1, *bufs[(c + 1) % 2])
        e_dma.wait()
        fc_dma.wait()
        compute(c, fc_c, rows_c)

    pltpu.sync_copy(out_v, out_hbm.at[pl.ds(wid * BPW, BPW)])


_fm_kernel = functools.partial(
    pl.kernel,
    out_type=jax.ShapeDtypeStruct((B,), jnp.float32),
    mesh=plsc.VectorSubcoreMesh(core_axis_name="c", subcore_axis_name="s"),
    compiler_params=pltpu.CompilerParams(
        needs_layout_passes=False, use_tc_tiling_on_sc=False),
    scratch_types=[
        pltpu.VMEM((BPW * F,), jnp.int32),
        pltpu.VMEM((G,), jnp.int32),
        pltpu.VMEM((G,), jnp.int32),
        pltpu.VMEM((G,), jnp.float32),
        pltpu.VMEM((G,), jnp.float32),
        pltpu.VMEM((G, K), jnp.float32),
        pltpu.VMEM((G, K), jnp.float32),
        pltpu.VMEM((C * K,), jnp.float32),
        pltpu.VMEM((BPW,), jnp.float32),
        pltpu.VMEM((L,), jnp.float32),
        pltpu.VMEM((L,), jnp.float32),
        pltpu.SemaphoreType.DMA,
        pltpu.SemaphoreType.DMA,
        pltpu.SemaphoreType.DMA,
        pltpu.SemaphoreType.DMA,
    ],
)(_fm_body)


def kernel(x, emb_table, fc_table, lin_w, lin_b, W0, b0, W1, b1, W2, b2):
    xf = x.reshape(B * F)
    emb_rm = _tr_kernel(emb_table.T).reshape(OUTR * 8, K)
    lw16 = jnp.broadcast_to(lin_w.reshape(1), (L,))
    lb16 = jnp.broadcast_to(lin_b.reshape(1), (L,))
    return _fm_kernel(xf, emb_rm, fc_table[:, 0], lw16, lb16)
